# Initial kernel scaffold; baseline (speedup 1.0000x reference)
import functools

import jax
import jax.numpy as jnp
from jax.experimental import pallas as pl
from jax.experimental.pallas import tpu as pltpu

INP_DIM = 512
EMB_DIM = 128
N_PROJ = 16
NUM_BINS = (64, 128, 256)
GROUP_W = tuple(128 * ((nb + 1 + 127) // 128) for nb in NUM_BINS)
K_TOT = N_PROJ * sum(GROUP_W)
BIG = 1e30


def _body(x_ref, proj_ref, glo_ref, ghi_ref, tbl_ref, out_ref, oh_ref):
    xb = x_ref[...]
    nrm = jnp.sqrt(jnp.sum(xb * xb, axis=1, keepdims=True))
    xn = xb / jnp.maximum(nrm, 1e-12)
    z = jnp.dot(xn, proj_ref[...], preferred_element_type=jnp.float32)
    bb = xb.shape[0]
    c0 = 0
    for s, nb in enumerate(NUM_BINS):
        w = GROUP_W[s]
        for p in range(N_PROJ):
            j = s * N_PROJ + p
            zc = jax.lax.broadcast_in_dim(z[:, j], (bb, w), (0,))
            lo = glo_ref[0:1, c0:c0 + w]
            hi = ghi_ref[0:1, c0:c0 + w]
            hit = (zc > lo) & (zc <= hi)
            oh_ref[:, c0:c0 + w] = jnp.where(
                hit, jnp.bfloat16(1.0), jnp.bfloat16(0.0))
            c0 += w
    out_ref[...] = jnp.dot(oh_ref[...], tbl_ref[...],
                           preferred_element_type=jnp.float32)


def kernel(x, proj0, grid0, table0, proj1, grid1, table1, proj2, grid2, table2):
    batch = x.shape[0]
    projcat = jnp.concatenate([proj0, proj1, proj2], axis=1)

    glo_parts, ghi_parts, tbl_parts = [], [], []
    for nb, w, grid, table in zip(
            NUM_BINS, GROUP_W, (grid0, grid1, grid2), (table0, table1, table2)):
        lo = jnp.concatenate([jnp.full((1,), -BIG, jnp.float32), grid,
                              jnp.full((w - nb - 1,), BIG, jnp.float32)])
        hi = jnp.concatenate([grid, jnp.full((w - nb,), BIG, jnp.float32)])
        glo_parts.append(jnp.tile(lo, (N_PROJ,)))
        ghi_parts.append(jnp.tile(hi, (N_PROJ,)))
        t = table.reshape(N_PROJ, nb + 1, EMB_DIM) * (1.0 / N_PROJ)
        t = jnp.pad(t, ((0, 0), (0, w - nb - 1), (0, 0)))
        tbl_parts.append(t.reshape(N_PROJ * w, EMB_DIM))
    glo = jnp.concatenate(glo_parts)[None, :]
    ghi = jnp.concatenate(ghi_parts)[None, :]
    tbl = jnp.concatenate(tbl_parts, axis=0).astype(jnp.bfloat16)

    bb = 256
    nblk = batch // bb
    grid_spec = pl.GridSpec(
        grid=(nblk,),
        in_specs=[
            pl.BlockSpec((bb, INP_DIM), lambda i: (i, 0)),
            pl.BlockSpec((INP_DIM, 48), lambda i: (0, 0)),
            pl.BlockSpec((1, K_TOT), lambda i: (0, 0)),
            pl.BlockSpec((1, K_TOT), lambda i: (0, 0)),
            pl.BlockSpec((K_TOT, EMB_DIM), lambda i: (0, 0)),
        ],
        out_specs=pl.BlockSpec((bb, EMB_DIM), lambda i: (i, 0)),
    )
    return pl.pallas_call(
        _body,
        grid_spec=grid_spec,
        out_shape=jax.ShapeDtypeStruct((batch, EMB_DIM), jnp.float32),
        scratch_shapes=[pltpu.VMEM((bb, K_TOT), jnp.bfloat16)],
    )(x, projcat, glo, ghi, tbl)


# speedup vs baseline: 151.8183x; 151.8183x over previous
import functools

import jax
import jax.numpy as jnp
from jax.experimental import pallas as pl
from jax.experimental.pallas import tpu as pltpu

INP_DIM = 512
EMB_DIM = 128
N_PROJ = 16
NUM_BINS = (64, 128, 256)
GROUP_W = tuple(128 * ((nb + 1 + 127) // 128) for nb in NUM_BINS)
K_TOT = N_PROJ * sum(GROUP_W)
BIG = 1e30


def _body(x_ref, proj_ref, glo_ref, ghi_ref, tbl_ref, out_ref, oh_ref):
    xb = x_ref[...]
    nrm = jnp.sqrt(jnp.sum(xb * xb, axis=1, keepdims=True))
    xn = xb / jnp.maximum(nrm, 1e-12)
    z = jnp.dot(xn, proj_ref[...], preferred_element_type=jnp.float32)
    bb = xb.shape[0]
    c0 = 0
    for s, nb in enumerate(NUM_BINS):
        w = GROUP_W[s]
        for p in range(N_PROJ):
            j = s * N_PROJ + p
            zc = jax.lax.broadcast_in_dim(z[:, j], (bb, w), (0,))
            lo = glo_ref[0:1, c0:c0 + w]
            hi = ghi_ref[0:1, c0:c0 + w]
            step_lo = jnp.where(zc > lo, 1.0, 0.0)
            step_hi = jnp.where(zc > hi, 1.0, 0.0)
            oh_ref[:, c0:c0 + w] = (step_lo - step_hi).astype(jnp.bfloat16)
            c0 += w
    out_ref[...] = jnp.dot(oh_ref[...], tbl_ref[...],
                           preferred_element_type=jnp.float32)


def kernel(x, proj0, grid0, table0, proj1, grid1, table1, proj2, grid2, table2):
    batch = x.shape[0]
    projcat = jnp.concatenate([proj0, proj1, proj2], axis=1)

    glo_parts, ghi_parts, tbl_parts = [], [], []
    for nb, w, grid, table in zip(
            NUM_BINS, GROUP_W, (grid0, grid1, grid2), (table0, table1, table2)):
        lo = jnp.concatenate([jnp.full((1,), -BIG, jnp.float32), grid,
                              jnp.full((w - nb - 1,), BIG, jnp.float32)])
        hi = jnp.concatenate([grid, jnp.full((w - nb,), BIG, jnp.float32)])
        glo_parts.append(jnp.tile(lo, (N_PROJ,)))
        ghi_parts.append(jnp.tile(hi, (N_PROJ,)))
        t = table.reshape(N_PROJ, nb + 1, EMB_DIM) * (1.0 / N_PROJ)
        t = jnp.pad(t, ((0, 0), (0, w - nb - 1), (0, 0)))
        tbl_parts.append(t.reshape(N_PROJ * w, EMB_DIM))
    glo = jnp.concatenate(glo_parts)[None, :]
    ghi = jnp.concatenate(ghi_parts)[None, :]
    tbl = jnp.concatenate(tbl_parts, axis=0).astype(jnp.bfloat16)

    bb = 256
    nblk = batch // bb
    return pl.pallas_call(
        _body,
        grid=(nblk,),
        in_specs=[
            pl.BlockSpec((bb, INP_DIM), lambda i: (i, 0)),
            pl.BlockSpec((INP_DIM, 48), lambda i: (0, 0)),
            pl.BlockSpec((1, K_TOT), lambda i: (0, 0)),
            pl.BlockSpec((1, K_TOT), lambda i: (0, 0)),
            pl.BlockSpec((K_TOT, EMB_DIM), lambda i: (0, 0)),
        ],
        out_specs=pl.BlockSpec((bb, EMB_DIM), lambda i: (i, 0)),
        out_shape=jax.ShapeDtypeStruct((batch, EMB_DIM), jnp.float32),
        scratch_shapes=[pltpu.VMEM((bb, K_TOT), jnp.bfloat16)],
    )(x, projcat, glo, ghi, tbl)
